# baseline (speedup 1.0000x reference)
import dataclasses
import functools

import jax
import jax.numpy as jnp
from jax import lax
from jax.experimental import pallas as pl
from jax.experimental.pallas import tpu as pltpu
from jax.experimental.pallas import tpu_sc as plsc

NC = 2
NS = 16
NW = NC * NS
L = 16
IDX_CHUNK = 128


@functools.lru_cache(maxsize=None)
def _make_sc_kernel(batch, d):
    per_w = batch // NW
    chunks = per_w // IDX_CHUNK
    mesh = plsc.VectorSubcoreMesh(core_axis_name="c", subcore_axis_name="s")
    cp = pltpu.CompilerParams(needs_layout_passes=False,
                              use_tc_tiling_on_sc=False)

    @functools.partial(
        pl.kernel,
        compiler_params=cp,
        out_type=jax.ShapeDtypeStruct((batch,), jnp.float32),
        mesh=mesh,
        scratch_types=[
            pltpu.VMEM((chunks, IDX_CHUNK), jnp.int32),
            pltpu.VMEM((chunks, IDX_CHUNK), jnp.int32),
            pltpu.VMEM((chunks, IDX_CHUNK), jnp.int32),
            pltpu.VMEM((per_w, d), jnp.float32),
            pltpu.VMEM((per_w, d), jnp.float32),
            pltpu.VMEM((per_w, d), jnp.float32),
            pltpu.VMEM((per_w * L,), jnp.float32),
            pltpu.VMEM((per_w,), jnp.float32),
            pltpu.SemaphoreType.DMA,
        ],
    )
    def k(head_hbm, rel_hbm, tail_hbm, ent_hbm, relemb_hbm, out_hbm,
          hi_v, ri_v, ti_v, h_v, r_v, t_v, p_v, o_v, sem):
        wid = lax.axis_index("s") * NC + lax.axis_index("c")
        base = wid * per_w

        for j in range(chunks):
            src = pl.ds(base + j * IDX_CHUNK, IDX_CHUNK)
            pltpu.sync_copy(head_hbm.at[src], hi_v.at[j])
            pltpu.sync_copy(rel_hbm.at[src], ri_v.at[j])
            pltpu.sync_copy(tail_hbm.at[src], ti_v.at[j])

        copies = []
        for j in range(chunks):
            dst = pl.ds(j * IDX_CHUNK, IDX_CHUNK)
            copies.append(pltpu.async_copy(ent_hbm.at[hi_v.at[j]], h_v.at[dst], sem))
            copies.append(pltpu.async_copy(relemb_hbm.at[ri_v.at[j]], r_v.at[dst], sem))
            copies.append(pltpu.async_copy(ent_hbm.at[ti_v.at[j]], t_v.at[dst], sem))
        for cp in copies:
            cp.wait()

        @pl.loop(0, per_w)
        def _(i):
            acc = h_v[i, pl.ds(0, L)] * r_v[i, pl.ds(0, L)] * t_v[i, pl.ds(0, L)]
            for c in range(1, d // L):
                sl = pl.ds(c * L, L)
                acc = acc + h_v[i, sl] * r_v[i, sl] * t_v[i, sl]
            p_v[pl.ds(i * L, L)] = acc

        iota = lax.iota(jnp.int32, L)

        @pl.loop(0, per_w // L)
        def _(g):
            bidx = g * (L * L) + iota * L
            acc = plsc.load_gather(p_v, [bidx])
            for kk in range(1, L):
                acc = acc + plsc.load_gather(p_v, [bidx + kk])
            o_v[pl.ds(g * L, L)] = acc

        pltpu.sync_copy(o_v, out_hbm.at[pl.ds(base, per_w)])

    return k


def kernel(head, relation, tail, entity_embeddings, relation_embeddings):
    batch = head.shape[0]
    d = entity_embeddings.shape[1]
    k = _make_sc_kernel(batch, d)
    return k(head.astype(jnp.int32), relation.astype(jnp.int32),
             tail.astype(jnp.int32), entity_embeddings, relation_embeddings)
